# baseline (speedup 1.0000x reference)
import functools

import jax
import jax.numpy as jnp
from jax import lax
from jax.experimental import pallas as pl
from jax.experimental.pallas import tpu as pltpu


def _select_body(tgt_ref, planes_ref, maskP_ref, idsP_ref, posP_ref,
                 mout_ref, *, A, NB, LB):
    t0, t1, t2 = tgt_ref[0], tgt_ref[1], tgt_ref[2]
    big = jnp.int32(A + 1)
    first = jnp.full((NB, LB), big, jnp.int32)
    for a in range(A):
        ids_a = idsP_ref[a]
        tm = (ids_a == t0) | (ids_a == t1) | (ids_a == t2)
        sel = maskP_ref[a] & tm[None, :]
        first = jnp.minimum(first, jnp.where(sel, jnp.int32(a), big))
    mout_ref[...] = (first < big).astype(jnp.float32)
    zero = jnp.zeros((NB, LB), jnp.float32)
    acc0, acc1, acc2 = zero, zero, zero
    for a in range(A):
        hit = first == a
        acc0 = jnp.where(hit, planes_ref[3 * a], acc0)
        acc1 = jnp.where(hit, planes_ref[3 * a + 1], acc1)
        acc2 = jnp.where(hit, planes_ref[3 * a + 2], acc2)
    posP_ref[0] = acc0
    posP_ref[1] = acc1
    posP_ref[2] = acc2


def kernel(pos_atoms, mask_atoms, atom_name_ids, target_ids):
    N, L, A, _ = pos_atoms.shape
    planes = pos_atoms.transpose(2, 3, 0, 1).reshape(3 * A, N, L)
    maskP = mask_atoms.transpose(2, 0, 1)
    idsP = atom_name_ids.T
    NB, LB = 8, 1024
    grid = (L // LB, N // NB)

    posP, mout = pl.pallas_call(
        functools.partial(_select_body, A=A, NB=NB, LB=LB),
        grid=grid,
        in_specs=[
            pl.BlockSpec(memory_space=pltpu.SMEM),
            pl.BlockSpec((3 * A, NB, LB), lambda jl, n: (0, n, jl)),
            pl.BlockSpec((A, NB, LB), lambda jl, n: (0, n, jl)),
            pl.BlockSpec((A, LB), lambda jl, n: (0, jl)),
        ],
        out_specs=[
            pl.BlockSpec((3, NB, LB), lambda jl, n: (0, n, jl)),
            pl.BlockSpec((NB, LB), lambda jl, n: (n, jl)),
        ],
        out_shape=[
            jax.ShapeDtypeStruct((3, N, L), jnp.float32),
            jax.ShapeDtypeStruct((N, L), jnp.float32),
        ],
        compiler_params=pltpu.CompilerParams(
            dimension_semantics=("parallel", "parallel"),
        ),
    )(target_ids, planes, maskP, idsP)

    return posP.transpose(1, 2, 0), mout


# speedup vs baseline: 1.0623x; 1.0623x over previous
import functools

import jax
import jax.numpy as jnp
from jax import lax
from jax.experimental import pallas as pl
from jax.experimental.pallas import tpu as pltpu


def _select_body(tgt_ref, planes_ref, maskP_ref, idsP_ref, posP_ref,
                 mout_ref, *, A, NB, LB):
    t0, t1, t2 = tgt_ref[0], tgt_ref[1], tgt_ref[2]
    big = jnp.int32(A + 1)
    first = jnp.full((NB, LB), big, jnp.int32)
    for a in range(A):
        ids_a = idsP_ref[a]
        tm = (ids_a == t0) | (ids_a == t1) | (ids_a == t2)
        sel = maskP_ref[a] & tm[None, :]
        first = jnp.minimum(first, jnp.where(sel, jnp.int32(a), big))
    mout_ref[...] = (first < big).astype(jnp.float32)
    zero = jnp.zeros((NB, LB), jnp.float32)
    acc0, acc1, acc2 = zero, zero, zero
    for a in range(A):
        hit = first == a
        acc0 = jnp.where(hit, planes_ref[3 * a], acc0)
        acc1 = jnp.where(hit, planes_ref[3 * a + 1], acc1)
        acc2 = jnp.where(hit, planes_ref[3 * a + 2], acc2)
    posP_ref[0] = acc0
    posP_ref[1] = acc1
    posP_ref[2] = acc2


def kernel(pos_atoms, mask_atoms, atom_name_ids, target_ids):
    N, L, A, _ = pos_atoms.shape
    planes = pos_atoms.transpose(2, 3, 0, 1).reshape(3 * A, N, L)
    maskP = mask_atoms.transpose(2, 0, 1)
    idsP = atom_name_ids.T
    NB, LB = 8, 2048
    grid = (L // LB, N // NB)

    posP, mout = pl.pallas_call(
        functools.partial(_select_body, A=A, NB=NB, LB=LB),
        grid=grid,
        in_specs=[
            pl.BlockSpec(memory_space=pltpu.SMEM),
            pl.BlockSpec((3 * A, NB, LB), lambda jl, n: (0, n, jl)),
            pl.BlockSpec((A, NB, LB), lambda jl, n: (0, n, jl)),
            pl.BlockSpec((A, LB), lambda jl, n: (0, jl)),
        ],
        out_specs=[
            pl.BlockSpec((3, NB, LB), lambda jl, n: (0, n, jl)),
            pl.BlockSpec((NB, LB), lambda jl, n: (n, jl)),
        ],
        out_shape=[
            jax.ShapeDtypeStruct((3, N, L), jnp.float32),
            jax.ShapeDtypeStruct((N, L), jnp.float32),
        ],
        compiler_params=pltpu.CompilerParams(
            dimension_semantics=("parallel", "parallel"),
        ),
    )(target_ids, planes, maskP, idsP)

    return posP.transpose(1, 2, 0), mout
